# Initial kernel scaffold; baseline (speedup 1.0000x reference)
import jax
import jax.numpy as jnp
from jax.experimental import pallas as pl


def kernel(x, edge_index, edge_attr, Wl1, Wr1, We1, att1, b1, Wl2, Wr2, We2, att2, b2, Wls, Wrs, Wes, atts, Wskip, bs, Wlin, blin, gamma, beta):
    raise NotImplementedError("write your pallas kernel here")


# speedup vs baseline: 16.4411x; 16.4411x over previous
import jax
import jax.numpy as jnp
from jax import lax
from jax.experimental import pallas as pl
from jax.experimental.pallas import tpu as pltpu
from jax.experimental.pallas import tpu_sc as plsc

N = 10000
E = 320000
H1, C1 = 8, 16
NC, NS, LANES = 2, 16, 16

W1 = 136
W2 = 48

_EPC = E // NC
_EPW = _EPC // NS
_K = 80
_NCHUNK = _EPW // _K
_RPS = N // NS


def _proj1_body(x_ref, wl_ref, wr_ref, ol_ref, or_ref):
    xb = x_ref[...]
    ol_ref[...] = jnp.dot(xb, wl_ref[...], preferred_element_type=jnp.float32)
    or_ref[...] = jnp.dot(xb, wr_ref[...], preferred_element_type=jnp.float32)


def _edge_proj_body(ea_ref, we1_ref, we2s_ref, o1_ref, o2_ref):
    eb = ea_ref[...]
    o1_ref[...] = jnp.dot(eb, we1_ref[...], preferred_element_type=jnp.float32)
    o2_ref[...] = jnp.dot(eb, we2s_ref[...], preferred_element_type=jnp.float32)


def _stage_c_body(p0_ref, p1_ref, xl1_ref, b1_ref, sel_ref, wsrc_ref,
                  wdst_ref, wskip_ref, tsrc_ref, tdst_ref):
    p0 = p0_ref[...]
    p1 = p1_ref[...]
    num = p0[:, 0:128] + p1[:, 0:128]
    den = p0[:, 128:136] + p1[:, 128:136]
    recb = jnp.dot(1.0 / (den + 1e-16), sel_ref[...],
                   preferred_element_type=jnp.float32)
    v = num * recb + b1_ref[...]
    h = jnp.where(v > 0, v, jnp.exp(v) - 1.0)
    sk = jnp.dot(xl1_ref[...], wskip_ref[...], preferred_element_type=jnp.float32)
    tsrc_ref[...] = jnp.concatenate(
        [jnp.dot(h, wsrc_ref[...], preferred_element_type=jnp.float32), sk], axis=1)
    tdst_ref[...] = jnp.dot(h, wdst_ref[...], preferred_element_type=jnp.float32)


def _stage_e_body(p0_ref, p1_ref, b2_ref, bs_ref, wlin_ref, blin_ref,
                  g_ref, be_ref, o_ref):
    q = p0_ref[...] + p1_ref[...]
    x1 = q[:, 0:16] / (q[:, 32:33] + 1e-16) + b2_ref[...]
    xs = q[:, 16:32] / (q[:, 33:34] + 1e-16) + bs_ref[...]
    x2 = jnp.dot(xs, wlin_ref[...], preferred_element_type=jnp.float32) + blin_ref[...]
    y = x1 + x2
    mu = jnp.mean(y, axis=-1, keepdims=True)
    var = jnp.mean((y - mu) ** 2, axis=-1, keepdims=True)
    o_ref[...] = (y - mu) * jax.lax.rsqrt(var + 1e-5) * g_ref[...] + be_ref[...]


def _sc_pipeline(src1_hbm, dst1_hbm, gat_tables, lin_hbm, zero_hbm, out_hbm,
                 idx_bufs, dsc, row_bufs, o_buf, acc, isems, gsem, ssem,
                 compute, zero_staging):
    cid = lax.axis_index("c")
    sid = lax.axis_index("s")

    pltpu.sync_copy(zero_hbm.at[pl.ds(sid * _RPS, _RPS)],
                    acc.at[pl.ds(sid * _RPS, _RPS)])
    for z in zero_staging:
        pltpu.sync_copy(zero_hbm.at[pl.ds(0, _K)], z)
    plsc.subcore_barrier()

    wbase = cid * _EPC + sid * _EPW
    kl = _NCHUNK - 1

    def idx_descs(k, s, make):
        si, di = idx_bufs[s]
        f = pltpu.make_async_copy if make else pltpu.async_copy
        return (f(src1_hbm.at[pl.ds(wbase + k * _K, _K)], si, isems[s]),
                f(dst1_hbm.at[pl.ds(wbase + k * _K, _K)], di, isems[s]))

    def idx_wait(k, s):
        for d in idx_descs(k, s, True):
            d.wait()

    def gather_descs(k, s, make):
        si, di = idx_bufs[s]
        s_r, d_r, e_r = row_bufs
        f = pltpu.make_async_copy if make else pltpu.async_copy
        return (f(gat_tables[0].at[si], s_r, gsem),
                f(gat_tables[1].at[di], d_r, gsem),
                f(lin_hbm.at[pl.ds(wbase + k * _K, _K)], e_r, gsem))

    def stage_scatter_idx(s):
        di = idx_bufs[s][1]
        for j in range(_K // LANES):
            dsc[pl.ds(j * LANES, LANES)] = di[pl.ds(j * LANES, LANES)]

    def scatter_issue():
        pltpu.async_copy(o_buf, acc.at[dsc], ssem, add=True)

    def scatter_wait():
        pltpu.make_async_copy(o_buf, acc.at[dsc], ssem).wait()

    def chunk(k, s, first):
        idx_wait(k, s)
        gather_descs(k, s, False)
        if not first:
            scatter_wait()
        for d in gather_descs(k, s, True):
            d.wait()
        stage_scatter_idx(s)
        idx_descs(k + 2, s, False)
        compute()
        scatter_issue()

    idx_descs(0, 0, False)
    idx_descs(1, 1, False)

    def loop_body(i, carry):
        idx_wait(2 * i, 0)
        gather_descs(2 * i, 0, False)

        @pl.when(i > 0)
        def _():
            scatter_wait()
        for d in gather_descs(2 * i, 0, True):
            d.wait()
        stage_scatter_idx(0)
        idx_descs(2 * i + 2, 0, False)
        compute()
        scatter_issue()

        idx_wait(2 * i + 1, 1)
        gather_descs(2 * i + 1, 1, False)
        scatter_wait()
        for d in gather_descs(2 * i + 1, 1, True):
            d.wait()
        stage_scatter_idx(1)

        @pl.when(i < (_NCHUNK - 1) // 2 - 1)
        def _():
            idx_descs(2 * i + 3, 1, False)
        compute()
        scatter_issue()
        return carry

    lax.fori_loop(0, (_NCHUNK - 1) // 2, loop_body, 0)
    idx_wait(kl, 0)
    gather_descs(kl, 0, False)
    scatter_wait()
    for d in gather_descs(kl, 0, True):
        d.wait()
    stage_scatter_idx(0)
    compute()
    scatter_issue()
    scatter_wait()
    plsc.subcore_barrier()
    pltpu.sync_copy(acc.at[pl.ds(sid * _RPS, _RPS)],
                    out_hbm.at[pl.ds(cid * N + sid * _RPS, _RPS)])


def _conv1_sc_body(xr_hbm, xl_hbm, e1_hbm, src1_hbm, dst1_hbm, attb_hbm,
                   zero_hbm, out_hbm,
                   si0, di0, si1, di1, dsc, xr0, xl0, e0,
                   o0, attb_v, acc, isem0, isem1, gsem, ssem):
    pltpu.sync_copy(attb_hbm, attb_v)

    def compute():
        xr_r, xl_r, e_r = xr0, xl0, e0
        o_r = o0

        def group_body(g, gcarry):
            eix = lax.iota(jnp.int32, LANES) + g * LANES
            for h in range(H1):
                acc_l = jnp.zeros((LANES,), jnp.float32)
                xr_cs = []
                for cc in range(C1):
                    c = h * C1 + cc
                    cf = jnp.full((LANES,), c, jnp.int32)
                    xr_c = plsc.load_gather(xr_r, [eix, cf])
                    xl_c = plsc.load_gather(xl_r, [eix, cf])
                    e_c = plsc.load_gather(e_r, [eix, cf])
                    m = xr_c + xl_c + e_c
                    leak = jnp.maximum(m, 0.2 * m)
                    acc_l = acc_l + leak * attb_v[c, :]
                    xr_cs.append(xr_c)
                w = jnp.exp(acc_l)
                for cc in range(C1):
                    cf = jnp.full((LANES,), h * C1 + cc, jnp.int32)
                    plsc.store_scatter(o_r, [eix, cf], xr_cs[cc] * w)
                plsc.store_scatter(o_r,
                                   [eix, jnp.full((LANES,), 128 + h, jnp.int32)],
                                   w)
            return gcarry

        lax.fori_loop(0, _K // LANES, group_body, 0)

    _sc_pipeline(src1_hbm, dst1_hbm, (xr_hbm, xl_hbm), e1_hbm, zero_hbm,
                 out_hbm, ((si0, di0), (si1, di1)), dsc, (xr0, xl0, e0),
                 o0, acc, (isem0, isem1), gsem, ssem, compute, ())


def _conv2_sc_body(tsrc_hbm, tdst_hbm, e2s_hbm, src1_hbm, dst1_hbm, attb_hbm,
                   zero_hbm, out_hbm,
                   si0, di0, si1, di1, dsc, s0, d0, e0,
                   o0, attb_v, acc, isem0, isem1, gsem, ssem):
    pltpu.sync_copy(attb_hbm, attb_v)

    def compute():
        s_r, d_r, e_r = s0, d0, e0
        o_r = o0

        def group_body(g, gcarry):
            eix = lax.iota(jnp.int32, LANES) + g * LANES
            acc2 = jnp.zeros((LANES,), jnp.float32)
            accs = jnp.zeros((LANES,), jnp.float32)
            xr2_cs = []
            xrs_cs = []
            for cc in range(16):
                cf0 = jnp.full((LANES,), cc, jnp.int32)
                cf1 = jnp.full((LANES,), 16 + cc, jnp.int32)
                cf2 = jnp.full((LANES,), 32 + cc, jnp.int32)
                xr2_c = plsc.load_gather(s_r, [eix, cf0])
                xrs_c = plsc.load_gather(s_r, [eix, cf1])
                sk_c = plsc.load_gather(s_r, [eix, cf2])
                xl2_c = plsc.load_gather(d_r, [eix, cf0])
                xls_c = plsc.load_gather(d_r, [eix, cf1])
                e2_c = plsc.load_gather(e_r, [eix, cf0])
                es_c = plsc.load_gather(e_r, [eix, cf1])
                m2 = xr2_c + xl2_c + e2_c
                ms = xrs_c + xls_c + es_c + sk_c
                l2 = jnp.maximum(m2, 0.2 * m2)
                ls = jnp.maximum(ms, 0.2 * ms)
                acc2 = acc2 + l2 * attb_v[cc, :]
                accs = accs + ls * attb_v[16 + cc, :]
                xr2_cs.append(xr2_c)
                xrs_cs.append(xrs_c)
            w2 = jnp.exp(acc2)
            ws = jnp.exp(accs)
            for cc in range(16):
                cf0 = jnp.full((LANES,), cc, jnp.int32)
                cf1 = jnp.full((LANES,), 16 + cc, jnp.int32)
                plsc.store_scatter(o_r, [eix, cf0], xr2_cs[cc] * w2)
                plsc.store_scatter(o_r, [eix, cf1], xrs_cs[cc] * ws)
            plsc.store_scatter(o_r,
                               [eix, jnp.full((LANES,), 32, jnp.int32)], w2)
            plsc.store_scatter(o_r,
                               [eix, jnp.full((LANES,), 33, jnp.int32)], ws)
            return gcarry

        lax.fori_loop(0, _K // LANES, group_body, 0)

    _sc_pipeline(src1_hbm, dst1_hbm, (tsrc_hbm, tdst_hbm), e2s_hbm, zero_hbm,
                 out_hbm, ((si0, di0), (si1, di1)), dsc, (s0, d0, e0),
                 o0, acc, (isem0, isem1), gsem, ssem, compute, (o0,))


def _sc_mesh():
    return plsc.VectorSubcoreMesh(core_axis_name="c", subcore_axis_name="s",
                                  num_cores=NC, num_subcores=NS)


_SC_PARAMS = pltpu.CompilerParams(needs_layout_passes=False,
                                  use_tc_tiling_on_sc=False)

_IDX_SCRATCH = [pltpu.VMEM((_K,), jnp.int32)] * 5
_SEM_SCRATCH = [pltpu.SemaphoreType.DMA] * 4

_conv1_sc = pl.kernel(
    _conv1_sc_body,
    out_type=jax.ShapeDtypeStruct((NC * N, W1), jnp.float32),
    mesh=_sc_mesh(),
    compiler_params=_SC_PARAMS,
    scratch_types=_IDX_SCRATCH + [
        pltpu.VMEM((_K, 128), jnp.float32),
        pltpu.VMEM((_K, 128), jnp.float32),
        pltpu.VMEM((_K, 128), jnp.float32),
        pltpu.VMEM((_K, W1), jnp.float32),
        pltpu.VMEM((128, 16), jnp.float32),
        pltpu.VMEM_SHARED((N, W1), jnp.float32),
    ] + _SEM_SCRATCH,
)

_conv2_sc = pl.kernel(
    _conv2_sc_body,
    out_type=jax.ShapeDtypeStruct((NC * N, W2), jnp.float32),
    mesh=_sc_mesh(),
    compiler_params=_SC_PARAMS,
    scratch_types=_IDX_SCRATCH + [
        pltpu.VMEM((_K, 48), jnp.float32),
        pltpu.VMEM((_K, 32), jnp.float32),
        pltpu.VMEM((_K, 32), jnp.float32),
        pltpu.VMEM((_K, W2), jnp.float32),
        pltpu.VMEM((32, 16), jnp.float32),
        pltpu.VMEM_SHARED((N, W2), jnp.float32),
    ] + _SEM_SCRATCH,
)


@jax.jit
def kernel(x, edge_index, edge_attr, Wl1, Wr1, We1, att1, b1, Wl2, Wr2, We2,
           att2, b2, Wls, Wrs, Wes, atts, Wskip, bs, Wlin, blin, gamma, beta):
    f32 = jnp.float32
    src1 = edge_index[0]
    dst1 = edge_index[1]

    xl1, xr1 = pl.pallas_call(
        _proj1_body,
        grid=(10,),
        in_specs=[
            pl.BlockSpec((1000, 128), lambda i: (i, 0)),
            pl.BlockSpec((128, 128), lambda i: (0, 0)),
            pl.BlockSpec((128, 128), lambda i: (0, 0)),
        ],
        out_specs=[
            pl.BlockSpec((1000, 128), lambda i: (i, 0)),
            pl.BlockSpec((1000, 128), lambda i: (i, 0)),
        ],
        out_shape=[
            jax.ShapeDtypeStruct((N, 128), f32),
            jax.ShapeDtypeStruct((N, 128), f32),
        ],
    )(x, Wl1, Wr1)

    We2s = jnp.concatenate([We2, Wes], axis=1)
    e1, e2s = pl.pallas_call(
        _edge_proj_body,
        grid=(320,),
        in_specs=[
            pl.BlockSpec((1000, 16), lambda i: (i, 0)),
            pl.BlockSpec((16, 128), lambda i: (0, 0)),
            pl.BlockSpec((16, 32), lambda i: (0, 0)),
        ],
        out_specs=[
            pl.BlockSpec((1000, 128), lambda i: (i, 0)),
            pl.BlockSpec((1000, 32), lambda i: (i, 0)),
        ],
        out_shape=[
            jax.ShapeDtypeStruct((E, 128), f32),
            jax.ShapeDtypeStruct((E, 32), f32),
        ],
    )(edge_attr, We1, We2s)

    attb1 = jnp.broadcast_to(att1.reshape(H1 * C1, 1), (H1 * C1, 16))
    zeros1 = jnp.zeros((N, W1), f32)
    p1 = _conv1_sc(xr1, xl1, e1, src1, dst1, attb1, zeros1)

    sel = jnp.repeat(jnp.eye(H1, dtype=f32), C1, axis=1)
    Wsrc = jnp.concatenate([Wr2, Wrs], axis=1)
    Wdst = jnp.concatenate([Wl2, Wls], axis=1)
    tsrc, tdst = pl.pallas_call(
        _stage_c_body,
        grid=(10,),
        in_specs=[
            pl.BlockSpec((1000, W1), lambda i: (i, 0)),
            pl.BlockSpec((1000, W1), lambda i: (i + 10, 0)),
            pl.BlockSpec((1000, 128), lambda i: (i, 0)),
            pl.BlockSpec((1, 128), lambda i: (0, 0)),
            pl.BlockSpec((8, 128), lambda i: (0, 0)),
            pl.BlockSpec((128, 32), lambda i: (0, 0)),
            pl.BlockSpec((128, 32), lambda i: (0, 0)),
            pl.BlockSpec((128, 16), lambda i: (0, 0)),
        ],
        out_specs=[
            pl.BlockSpec((1000, 48), lambda i: (i, 0)),
            pl.BlockSpec((1000, 32), lambda i: (i, 0)),
        ],
        out_shape=[
            jax.ShapeDtypeStruct((N, 48), f32),
            jax.ShapeDtypeStruct((N, 32), f32),
        ],
    )(p1, p1, xl1, b1.reshape(1, 128), sel, Wsrc, Wdst, Wskip)

    attb2 = jnp.broadcast_to(
        jnp.concatenate([att2.reshape(16), atts.reshape(16)]).reshape(32, 1),
        (32, 16))
    zeros2 = jnp.zeros((N, W2), f32)
    p2 = _conv2_sc(tsrc, tdst, e2s, src1, dst1, attb2, zeros2)

    y = pl.pallas_call(
        _stage_e_body,
        grid=(10,),
        in_specs=[
            pl.BlockSpec((1000, W2), lambda i: (i, 0)),
            pl.BlockSpec((1000, W2), lambda i: (i + 10, 0)),
            pl.BlockSpec((1, 16), lambda i: (0, 0)),
            pl.BlockSpec((1, 16), lambda i: (0, 0)),
            pl.BlockSpec((16, 16), lambda i: (0, 0)),
            pl.BlockSpec((1, 16), lambda i: (0, 0)),
            pl.BlockSpec((1, 16), lambda i: (0, 0)),
            pl.BlockSpec((1, 16), lambda i: (0, 0)),
        ],
        out_specs=pl.BlockSpec((1000, 16), lambda i: (i, 0)),
        out_shape=jax.ShapeDtypeStruct((N, 16), f32),
    )(p2, p2, b2.reshape(1, 16), bs.reshape(1, 16), Wlin,
      blin.reshape(1, 16), gamma.reshape(1, 16), beta.reshape(1, 16))

    return jnp.stack((y,))


# speedup vs baseline: 16.5793x; 1.0084x over previous
import jax
import jax.numpy as jnp
from jax import lax
from jax.experimental import pallas as pl
from jax.experimental.pallas import tpu as pltpu
from jax.experimental.pallas import tpu_sc as plsc

N = 10000
E = 320000
H1, C1 = 8, 16
NC, NS, LANES = 2, 16, 16

W1 = 136
W2 = 48

_EPC = E // NC
_EPW = _EPC // NS
_K = 80
_NCHUNK = _EPW // _K
_RPS = N // NS


def _proj1_body(x_ref, wl_ref, wr_ref, ol_ref, or_ref):
    xb = x_ref[...]
    ol_ref[...] = jnp.dot(
        xb, wl_ref[...], preferred_element_type=jnp.float32).astype(jnp.bfloat16)
    or_ref[...] = jnp.dot(
        xb, wr_ref[...], preferred_element_type=jnp.float32).astype(jnp.bfloat16)


def _edge_proj_body(ea_ref, we1_ref, we2s_ref, o1_ref, o2_ref):
    eb = ea_ref[...]
    o1_ref[...] = jnp.dot(
        eb, we1_ref[...], preferred_element_type=jnp.float32).astype(jnp.bfloat16)
    o2_ref[...] = jnp.dot(
        eb, we2s_ref[...], preferred_element_type=jnp.float32).astype(jnp.bfloat16)


def _stage_c_body(p0_ref, p1_ref, xl1_ref, b1_ref, sel_ref, wsrc_ref,
                  wdst_ref, wskip_ref, tsrc_ref, tdst_ref):
    p0 = p0_ref[...]
    p1 = p1_ref[...]
    num = p0[:, 0:128] + p1[:, 0:128]
    den = p0[:, 128:136] + p1[:, 128:136]
    recb = jnp.dot(1.0 / (den + 1e-16), sel_ref[...],
                   preferred_element_type=jnp.float32)
    v = num * recb + b1_ref[...]
    h = jnp.where(v > 0, v, jnp.exp(v) - 1.0)
    sk = jnp.dot(xl1_ref[...], wskip_ref[...], preferred_element_type=jnp.float32)
    pad = jnp.zeros_like(sk)
    tsrc_ref[...] = jnp.concatenate(
        [jnp.dot(h, wsrc_ref[...], preferred_element_type=jnp.float32), sk, pad],
        axis=1).astype(jnp.bfloat16)
    tdst_ref[...] = jnp.dot(
        h, wdst_ref[...], preferred_element_type=jnp.float32).astype(jnp.bfloat16)


def _stage_e_body(p0_ref, p1_ref, b2_ref, bs_ref, wlin_ref, blin_ref,
                  g_ref, be_ref, o_ref):
    q = p0_ref[...] + p1_ref[...]
    x1 = q[:, 0:16] / (q[:, 32:33] + 1e-16) + b2_ref[...]
    xs = q[:, 16:32] / (q[:, 33:34] + 1e-16) + bs_ref[...]
    x2 = jnp.dot(xs, wlin_ref[...], preferred_element_type=jnp.float32) + blin_ref[...]
    y = x1 + x2
    mu = jnp.mean(y, axis=-1, keepdims=True)
    var = jnp.mean((y - mu) ** 2, axis=-1, keepdims=True)
    o_ref[...] = (y - mu) * jax.lax.rsqrt(var + 1e-5) * g_ref[...] + be_ref[...]


def _sc_pipeline(src1_hbm, dst1_hbm, gat_tables, lin_hbm, zero_hbm, out_hbm,
                 idx_bufs, dsc, row_bufs, o_buf, acc, isems, gsem, ssem,
                 compute, zero_staging):
    cid = lax.axis_index("c")
    sid = lax.axis_index("s")

    pltpu.sync_copy(zero_hbm.at[pl.ds(sid * _RPS, _RPS)],
                    acc.at[pl.ds(sid * _RPS, _RPS)])
    for z in zero_staging:
        pltpu.sync_copy(zero_hbm.at[pl.ds(0, _K)], z)
    plsc.subcore_barrier()

    wbase = cid * _EPC + sid * _EPW
    kl = _NCHUNK - 1

    def idx_descs(k, s, make):
        si, di = idx_bufs[s]
        f = pltpu.make_async_copy if make else pltpu.async_copy
        return (f(src1_hbm.at[pl.ds(wbase + k * _K, _K)], si, isems[s]),
                f(dst1_hbm.at[pl.ds(wbase + k * _K, _K)], di, isems[s]))

    def idx_wait(k, s):
        for d in idx_descs(k, s, True):
            d.wait()

    def gather_descs(k, s, make):
        si, di = idx_bufs[s]
        s_r, d_r, e_r = row_bufs
        f = pltpu.make_async_copy if make else pltpu.async_copy
        return (f(gat_tables[0].at[si], s_r, gsem),
                f(gat_tables[1].at[di], d_r, gsem),
                f(lin_hbm.at[pl.ds(wbase + k * _K, _K)], e_r, gsem))

    def stage_scatter_idx(s):
        di = idx_bufs[s][1]
        for j in range(_K // LANES):
            dsc[pl.ds(j * LANES, LANES)] = di[pl.ds(j * LANES, LANES)]

    def scatter_issue():
        pltpu.async_copy(o_buf, acc.at[dsc], ssem, add=True)

    def scatter_wait():
        pltpu.make_async_copy(o_buf, acc.at[dsc], ssem).wait()

    def chunk(k, s, first):
        idx_wait(k, s)
        gather_descs(k, s, False)
        if not first:
            scatter_wait()
        for d in gather_descs(k, s, True):
            d.wait()
        stage_scatter_idx(s)
        idx_descs(k + 2, s, False)
        compute()
        scatter_issue()

    idx_descs(0, 0, False)
    idx_descs(1, 1, False)

    def loop_body(i, carry):
        idx_wait(2 * i, 0)
        gather_descs(2 * i, 0, False)

        @pl.when(i > 0)
        def _():
            scatter_wait()
        for d in gather_descs(2 * i, 0, True):
            d.wait()
        stage_scatter_idx(0)
        idx_descs(2 * i + 2, 0, False)
        compute()
        scatter_issue()

        idx_wait(2 * i + 1, 1)
        gather_descs(2 * i + 1, 1, False)
        scatter_wait()
        for d in gather_descs(2 * i + 1, 1, True):
            d.wait()
        stage_scatter_idx(1)

        @pl.when(i < (_NCHUNK - 1) // 2 - 1)
        def _():
            idx_descs(2 * i + 3, 1, False)
        compute()
        scatter_issue()
        return carry

    lax.fori_loop(0, (_NCHUNK - 1) // 2, loop_body, 0)
    idx_wait(kl, 0)
    gather_descs(kl, 0, False)
    scatter_wait()
    for d in gather_descs(kl, 0, True):
        d.wait()
    stage_scatter_idx(0)
    compute()
    scatter_issue()
    scatter_wait()
    plsc.subcore_barrier()
    pltpu.sync_copy(acc.at[pl.ds(sid * _RPS, _RPS)],
                    out_hbm.at[pl.ds(cid * N + sid * _RPS, _RPS)])


def _conv1_sc_body(xr_hbm, xl_hbm, e1_hbm, src1_hbm, dst1_hbm, attb_hbm,
                   zero_hbm, out_hbm,
                   si0, di0, si1, di1, dsc, xr0, xl0, e0,
                   o0, attb_v, acc, isem0, isem1, gsem, ssem):
    pltpu.sync_copy(attb_hbm, attb_v)

    def compute():
        xr_r, xl_r, e_r = xr0, xl0, e0
        o_r = o0

        def unpk(ref, eix, wf):
            word = plsc.load_gather(ref, [eix, wf])
            return plsc.unpack(plsc.bitcast(word, jnp.bfloat16),
                               format=plsc.PackFormat.INTERLEAVED)

        def group_body(g, gcarry):
            eix = lax.iota(jnp.int32, LANES) + g * LANES
            for h in range(H1):
                acc_l = jnp.zeros((LANES,), jnp.float32)
                xr_cs = []
                for pp in range(C1 // 2):
                    wf = jnp.full((LANES,), h * (C1 // 2) + pp, jnp.int32)
                    xr_ab = unpk(xr_r, eix, wf)
                    xl_ab = unpk(xl_r, eix, wf)
                    e_ab = unpk(e_r, eix, wf)
                    for q in range(2):
                        c = h * C1 + 2 * pp + q
                        m = xr_ab[q] + xl_ab[q] + e_ab[q]
                        leak = jnp.maximum(m, 0.2 * m)
                        acc_l = acc_l + leak * attb_v[c, :]
                        xr_cs.append(xr_ab[q])
                w = jnp.exp(acc_l)
                for cc in range(C1):
                    cf = jnp.full((LANES,), h * C1 + cc, jnp.int32)
                    plsc.store_scatter(o_r, [eix, cf], xr_cs[cc] * w)
                plsc.store_scatter(o_r,
                                   [eix, jnp.full((LANES,), 128 + h, jnp.int32)],
                                   w)
            return gcarry

        lax.fori_loop(0, _K // LANES, group_body, 0)

    _sc_pipeline(src1_hbm, dst1_hbm, (xr_hbm, xl_hbm), e1_hbm, zero_hbm,
                 out_hbm, ((si0, di0), (si1, di1)), dsc, (xr0, xl0, e0),
                 o0, acc, (isem0, isem1), gsem, ssem, compute, ())


def _conv2_sc_body(tsrc_hbm, tdst_hbm, e2s_hbm, src1_hbm, dst1_hbm, attb_hbm,
                   zero_hbm, out_hbm,
                   si0, di0, si1, di1, dsc, s0, d0, e0,
                   o0, attb_v, acc, isem0, isem1, gsem, ssem):
    pltpu.sync_copy(attb_hbm, attb_v)

    def compute():
        s_r, d_r, e_r = s0, d0, e0
        o_r = o0

        def unpk(ref, eix, wi):
            word = plsc.load_gather(ref, [eix, jnp.full((LANES,), wi, jnp.int32)])
            return plsc.unpack(plsc.bitcast(word, jnp.bfloat16),
                               format=plsc.PackFormat.INTERLEAVED)

        def group_body(g, gcarry):
            eix = lax.iota(jnp.int32, LANES) + g * LANES
            acc2 = jnp.zeros((LANES,), jnp.float32)
            accs = jnp.zeros((LANES,), jnp.float32)
            xr2_cs = []
            xrs_cs = []
            for pp in range(8):
                xr2_ab = unpk(s_r, eix, pp)
                xrs_ab = unpk(s_r, eix, 8 + pp)
                sk_ab = unpk(s_r, eix, 16 + pp)
                xl2_ab = unpk(d_r, eix, pp)
                xls_ab = unpk(d_r, eix, 8 + pp)
                e2_ab = unpk(e_r, eix, pp)
                es_ab = unpk(e_r, eix, 8 + pp)
                for q in range(2):
                    cc = 2 * pp + q
                    m2 = xr2_ab[q] + xl2_ab[q] + e2_ab[q]
                    ms = xrs_ab[q] + xls_ab[q] + es_ab[q] + sk_ab[q]
                    l2 = jnp.maximum(m2, 0.2 * m2)
                    ls = jnp.maximum(ms, 0.2 * ms)
                    acc2 = acc2 + l2 * attb_v[cc, :]
                    accs = accs + ls * attb_v[16 + cc, :]
                    xr2_cs.append(xr2_ab[q])
                    xrs_cs.append(xrs_ab[q])
            w2 = jnp.exp(acc2)
            ws = jnp.exp(accs)
            for cc in range(16):
                cf0 = jnp.full((LANES,), cc, jnp.int32)
                cf1 = jnp.full((LANES,), 16 + cc, jnp.int32)
                plsc.store_scatter(o_r, [eix, cf0], xr2_cs[cc] * w2)
                plsc.store_scatter(o_r, [eix, cf1], xrs_cs[cc] * ws)
            plsc.store_scatter(o_r,
                               [eix, jnp.full((LANES,), 32, jnp.int32)], w2)
            plsc.store_scatter(o_r,
                               [eix, jnp.full((LANES,), 33, jnp.int32)], ws)
            return gcarry

        lax.fori_loop(0, _K // LANES, group_body, 0)

    _sc_pipeline(src1_hbm, dst1_hbm, (tsrc_hbm, tdst_hbm), e2s_hbm, zero_hbm,
                 out_hbm, ((si0, di0), (si1, di1)), dsc, (s0, d0, e0),
                 o0, acc, (isem0, isem1), gsem, ssem, compute, (o0,))


def _sc_mesh():
    return plsc.VectorSubcoreMesh(core_axis_name="c", subcore_axis_name="s",
                                  num_cores=NC, num_subcores=NS)


_SC_PARAMS = pltpu.CompilerParams(needs_layout_passes=False,
                                  use_tc_tiling_on_sc=False)

_IDX_SCRATCH = [pltpu.VMEM((_K,), jnp.int32)] * 5
_SEM_SCRATCH = [pltpu.SemaphoreType.DMA] * 4

_conv1_sc = pl.kernel(
    _conv1_sc_body,
    out_type=jax.ShapeDtypeStruct((NC * N, W1), jnp.float32),
    mesh=_sc_mesh(),
    compiler_params=_SC_PARAMS,
    scratch_types=_IDX_SCRATCH + [
        pltpu.VMEM((_K, 64), jnp.int32),
        pltpu.VMEM((_K, 64), jnp.int32),
        pltpu.VMEM((_K, 64), jnp.int32),
        pltpu.VMEM((_K, W1), jnp.float32),
        pltpu.VMEM((128, 16), jnp.float32),
        pltpu.VMEM_SHARED((N, W1), jnp.float32),
    ] + _SEM_SCRATCH,
)

_conv2_sc = pl.kernel(
    _conv2_sc_body,
    out_type=jax.ShapeDtypeStruct((NC * N, W2), jnp.float32),
    mesh=_sc_mesh(),
    compiler_params=_SC_PARAMS,
    scratch_types=_IDX_SCRATCH + [
        pltpu.VMEM((_K, 32), jnp.int32),
        pltpu.VMEM((_K, 16), jnp.int32),
        pltpu.VMEM((_K, 16), jnp.int32),
        pltpu.VMEM((_K, W2), jnp.float32),
        pltpu.VMEM((32, 16), jnp.float32),
        pltpu.VMEM_SHARED((N, W2), jnp.float32),
    ] + _SEM_SCRATCH,
)


@jax.jit
def kernel(x, edge_index, edge_attr, Wl1, Wr1, We1, att1, b1, Wl2, Wr2, We2,
           att2, b2, Wls, Wrs, Wes, atts, Wskip, bs, Wlin, blin, gamma, beta):
    f32 = jnp.float32
    src1 = edge_index[0]
    dst1 = edge_index[1]

    xl1, xr1 = pl.pallas_call(
        _proj1_body,
        grid=(10,),
        in_specs=[
            pl.BlockSpec((1000, 128), lambda i: (i, 0)),
            pl.BlockSpec((128, 128), lambda i: (0, 0)),
            pl.BlockSpec((128, 128), lambda i: (0, 0)),
        ],
        out_specs=[
            pl.BlockSpec((1000, 128), lambda i: (i, 0)),
            pl.BlockSpec((1000, 128), lambda i: (i, 0)),
        ],
        out_shape=[
            jax.ShapeDtypeStruct((N, 128), jnp.bfloat16),
            jax.ShapeDtypeStruct((N, 128), jnp.bfloat16),
        ],
    )(x, Wl1, Wr1)
    xl1i = jax.lax.bitcast_convert_type(
        xl1.reshape(N, 64, 2), jnp.int32)
    xr1i = jax.lax.bitcast_convert_type(
        xr1.reshape(N, 64, 2), jnp.int32)

    We2s = jnp.concatenate([We2, Wes], axis=1)
    e1, e2s = pl.pallas_call(
        _edge_proj_body,
        grid=(320,),
        in_specs=[
            pl.BlockSpec((1000, 16), lambda i: (i, 0)),
            pl.BlockSpec((16, 128), lambda i: (0, 0)),
            pl.BlockSpec((16, 32), lambda i: (0, 0)),
        ],
        out_specs=[
            pl.BlockSpec((1000, 128), lambda i: (i, 0)),
            pl.BlockSpec((1000, 32), lambda i: (i, 0)),
        ],
        out_shape=[
            jax.ShapeDtypeStruct((E, 128), jnp.bfloat16),
            jax.ShapeDtypeStruct((E, 32), jnp.bfloat16),
        ],
    )(edge_attr, We1, We2s)
    e1i = jax.lax.bitcast_convert_type(e1.reshape(E, 64, 2), jnp.int32)
    e2si = jax.lax.bitcast_convert_type(e2s.reshape(E, 16, 2), jnp.int32)

    attb1 = jnp.broadcast_to(att1.reshape(H1 * C1, 1), (H1 * C1, 16))
    zeros1 = jnp.zeros((N, W1), f32)
    p1 = _conv1_sc(xr1i, xl1i, e1i, src1, dst1, attb1, zeros1)

    sel = jnp.repeat(jnp.eye(H1, dtype=f32), C1, axis=1)
    Wsrc = jnp.concatenate([Wr2, Wrs], axis=1)
    Wdst = jnp.concatenate([Wl2, Wls], axis=1)
    tsrc, tdst = pl.pallas_call(
        _stage_c_body,
        grid=(10,),
        in_specs=[
            pl.BlockSpec((1000, W1), lambda i: (i, 0)),
            pl.BlockSpec((1000, W1), lambda i: (i + 10, 0)),
            pl.BlockSpec((1000, 128), lambda i: (i, 0)),
            pl.BlockSpec((1, 128), lambda i: (0, 0)),
            pl.BlockSpec((8, 128), lambda i: (0, 0)),
            pl.BlockSpec((128, 32), lambda i: (0, 0)),
            pl.BlockSpec((128, 32), lambda i: (0, 0)),
            pl.BlockSpec((128, 16), lambda i: (0, 0)),
        ],
        out_specs=[
            pl.BlockSpec((1000, 64), lambda i: (i, 0)),
            pl.BlockSpec((1000, 32), lambda i: (i, 0)),
        ],
        out_shape=[
            jax.ShapeDtypeStruct((N, 64), jnp.bfloat16),
            jax.ShapeDtypeStruct((N, 32), jnp.bfloat16),
        ],
    )(p1, p1, xl1, b1.reshape(1, 128), sel, Wsrc, Wdst, Wskip)
    tsrci = jax.lax.bitcast_convert_type(tsrc.reshape(N, 32, 2), jnp.int32)
    tdsti = jax.lax.bitcast_convert_type(tdst.reshape(N, 16, 2), jnp.int32)

    attb2 = jnp.broadcast_to(
        jnp.concatenate([att2.reshape(16), atts.reshape(16)]).reshape(32, 1),
        (32, 16))
    zeros2 = jnp.zeros((N, W2), f32)
    p2 = _conv2_sc(tsrci, tdsti, e2si, src1, dst1, attb2, zeros2)

    y = pl.pallas_call(
        _stage_e_body,
        grid=(10,),
        in_specs=[
            pl.BlockSpec((1000, W2), lambda i: (i, 0)),
            pl.BlockSpec((1000, W2), lambda i: (i + 10, 0)),
            pl.BlockSpec((1, 16), lambda i: (0, 0)),
            pl.BlockSpec((1, 16), lambda i: (0, 0)),
            pl.BlockSpec((16, 16), lambda i: (0, 0)),
            pl.BlockSpec((1, 16), lambda i: (0, 0)),
            pl.BlockSpec((1, 16), lambda i: (0, 0)),
            pl.BlockSpec((1, 16), lambda i: (0, 0)),
        ],
        out_specs=pl.BlockSpec((1000, 16), lambda i: (i, 0)),
        out_shape=jax.ShapeDtypeStruct((N, 16), f32),
    )(p2, p2, b2.reshape(1, 16), bs.reshape(1, 16), Wlin,
      blin.reshape(1, 16), gamma.reshape(1, 16), beta.reshape(1, 16))

    return jnp.stack((y,))


# speedup vs baseline: 24.3102x; 1.4663x over previous
import jax
import jax.numpy as jnp
from jax import lax
from jax.experimental import pallas as pl
from jax.experimental.pallas import tpu as pltpu
from jax.experimental.pallas import tpu_sc as plsc

N = 10000
E = 320000
H1, C1 = 8, 16
NC, NS, LANES = 2, 16, 16

W1 = 136
W2 = 48

_EPC = E // NC
_EPW = _EPC // NS
_K = 80
_NCHUNK = _EPW // _K
_RPS = N // NS


def _pack_i32(v):
    u = jax.lax.bitcast_convert_type(v, jnp.uint32) + jnp.uint32(0x8000)
    w = v.shape[1] // 2
    word = (u[:, :w] >> 16) | (u[:, w:] & jnp.uint32(0xFFFF0000))
    return jax.lax.bitcast_convert_type(word, jnp.int32)


def _proj1_body(x_ref, wl_ref, wr_ref, ol_ref, or_ref):
    xb = x_ref[...]
    ol_ref[...] = _pack_i32(
        jnp.dot(xb, wl_ref[...], preferred_element_type=jnp.float32))
    or_ref[...] = _pack_i32(
        jnp.dot(xb, wr_ref[...], preferred_element_type=jnp.float32))


def _edge_proj_body(ea_ref, we1_ref, we2s_ref, o1_ref, o2_ref):
    eb = ea_ref[...]
    o1_ref[...] = _pack_i32(
        jnp.dot(eb, we1_ref[...], preferred_element_type=jnp.float32))
    o2_ref[...] = _pack_i32(
        jnp.dot(eb, we2s_ref[...], preferred_element_type=jnp.float32))


def _stage_c_body(p0_ref, p1_ref, xl1_ref, b1_ref, sel_ref, wsrc_ref,
                  wdst_ref, wskip_ref, tsrc_ref, tdst_ref):
    u = jax.lax.bitcast_convert_type(xl1_ref[...], jnp.uint32)
    xl1 = jnp.concatenate(
        [jax.lax.bitcast_convert_type(u << 16, jnp.float32),
         jax.lax.bitcast_convert_type(u & jnp.uint32(0xFFFF0000), jnp.float32)],
        axis=1)
    p0 = p0_ref[...]
    p1 = p1_ref[...]
    num = p0[:, 0:128] + p1[:, 0:128]
    den = p0[:, 128:136] + p1[:, 128:136]
    recb = jnp.dot(1.0 / (den + 1e-16), sel_ref[...],
                   preferred_element_type=jnp.float32)
    v = num * recb + b1_ref[...]
    h = jnp.where(v > 0, v, jnp.exp(v) - 1.0)
    sk = jnp.dot(xl1, wskip_ref[...], preferred_element_type=jnp.float32)
    pad = jnp.zeros_like(sk)
    tsrc_ref[...] = _pack_i32(jnp.concatenate(
        [jnp.dot(h, wsrc_ref[...], preferred_element_type=jnp.float32), sk, pad],
        axis=1))
    tdst_ref[...] = _pack_i32(
        jnp.dot(h, wdst_ref[...], preferred_element_type=jnp.float32))


def _stage_e_body(p0_ref, p1_ref, b2_ref, bs_ref, wlin_ref, blin_ref,
                  g_ref, be_ref, o_ref):
    q = p0_ref[...] + p1_ref[...]
    x1 = q[:, 0:16] / (q[:, 32:33] + 1e-16) + b2_ref[...]
    xs = q[:, 16:32] / (q[:, 33:34] + 1e-16) + bs_ref[...]
    x2 = jnp.dot(xs, wlin_ref[...], preferred_element_type=jnp.float32) + blin_ref[...]
    y = x1 + x2
    mu = jnp.mean(y, axis=-1, keepdims=True)
    var = jnp.mean((y - mu) ** 2, axis=-1, keepdims=True)
    o_ref[...] = (y - mu) * jax.lax.rsqrt(var + 1e-5) * g_ref[...] + be_ref[...]


def _sc_pipeline(src1_hbm, dst1_hbm, gat_tables, lin_hbm, zero_hbm, out_hbm,
                 idx_bufs, dsc, row_bufs, o_buf, acc, isems, gsem, ssem,
                 compute, zero_staging):
    cid = lax.axis_index("c")
    sid = lax.axis_index("s")

    pltpu.sync_copy(zero_hbm.at[pl.ds(sid * _RPS, _RPS)],
                    acc.at[pl.ds(sid * _RPS, _RPS)])
    for z in zero_staging:
        pltpu.sync_copy(zero_hbm.at[pl.ds(0, _K)], z)
    plsc.subcore_barrier()

    wbase = cid * _EPC + sid * _EPW
    kl = _NCHUNK - 1

    def idx_descs(k, s, make):
        si, di = idx_bufs[s]
        f = pltpu.make_async_copy if make else pltpu.async_copy
        return (f(src1_hbm.at[pl.ds(wbase + k * _K, _K)], si, isems[s]),
                f(dst1_hbm.at[pl.ds(wbase + k * _K, _K)], di, isems[s]))

    def idx_wait(k, s):
        for d in idx_descs(k, s, True):
            d.wait()

    def gather_descs(k, s, make):
        si, di = idx_bufs[s]
        s_r, d_r, e_r = row_bufs
        f = pltpu.make_async_copy if make else pltpu.async_copy
        return (f(gat_tables[0].at[si], s_r, gsem),
                f(gat_tables[1].at[di], d_r, gsem),
                f(lin_hbm.at[pl.ds(wbase + k * _K, _K)], e_r, gsem))

    def stage_scatter_idx(s):
        di = idx_bufs[s][1]
        for j in range(_K // LANES):
            dsc[pl.ds(j * LANES, LANES)] = di[pl.ds(j * LANES, LANES)]

    def scatter_issue():
        pltpu.async_copy(o_buf, acc.at[dsc], ssem, add=True)

    def scatter_wait():
        pltpu.make_async_copy(o_buf, acc.at[dsc], ssem).wait()

    def chunk(k, s, first):
        idx_wait(k, s)
        gather_descs(k, s, False)
        if not first:
            scatter_wait()
        for d in gather_descs(k, s, True):
            d.wait()
        stage_scatter_idx(s)
        idx_descs(k + 2, s, False)
        compute()
        scatter_issue()

    idx_descs(0, 0, False)
    idx_descs(1, 1, False)

    def loop_body(i, carry):
        idx_wait(2 * i, 0)
        gather_descs(2 * i, 0, False)

        @pl.when(i > 0)
        def _():
            scatter_wait()
        for d in gather_descs(2 * i, 0, True):
            d.wait()
        stage_scatter_idx(0)
        idx_descs(2 * i + 2, 0, False)
        compute()
        scatter_issue()

        idx_wait(2 * i + 1, 1)
        gather_descs(2 * i + 1, 1, False)
        scatter_wait()
        for d in gather_descs(2 * i + 1, 1, True):
            d.wait()
        stage_scatter_idx(1)

        @pl.when(i < (_NCHUNK - 1) // 2 - 1)
        def _():
            idx_descs(2 * i + 3, 1, False)
        compute()
        scatter_issue()
        return carry

    lax.fori_loop(0, (_NCHUNK - 1) // 2, loop_body, 0)
    idx_wait(kl, 0)
    gather_descs(kl, 0, False)
    scatter_wait()
    for d in gather_descs(kl, 0, True):
        d.wait()
    stage_scatter_idx(0)
    compute()
    scatter_issue()
    scatter_wait()
    plsc.subcore_barrier()
    pltpu.sync_copy(acc.at[pl.ds(sid * _RPS, _RPS)],
                    out_hbm.at[pl.ds(cid * N + sid * _RPS, _RPS)])


def _conv1_sc_body(xr_hbm, xl_hbm, e1_hbm, src1_hbm, dst1_hbm, attb_hbm,
                   zero_hbm, out_hbm,
                   si0, di0, si1, di1, dsc, xr0, xl0, e0,
                   o0, attb_v, acc, isem0, isem1, gsem, ssem):
    pltpu.sync_copy(attb_hbm, attb_v)

    def compute():
        xr_r, xl_r, e_r = xr0, xl0, e0
        o_r = o0

        def unpk(ref, eix, wi):
            word = plsc.load_gather(ref, [eix, jnp.full((LANES,), wi, jnp.int32)])
            return plsc.unpack(plsc.bitcast(word, jnp.bfloat16),
                               format=plsc.PackFormat.INTERLEAVED)

        def group_body(g, gcarry):
            eix = lax.iota(jnp.int32, LANES) + g * LANES
            for hp in range(H1 // 2):
                a_lo = jnp.zeros((LANES,), jnp.float32)
                a_hi = jnp.zeros((LANES,), jnp.float32)
                xr_lo = []
                xr_hi = []
                for cc in range(C1):
                    wi = hp * C1 + cc
                    xr_ab = unpk(xr_r, eix, wi)
                    xl_ab = unpk(xl_r, eix, wi)
                    e_ab = unpk(e_r, eix, wi)
                    m0 = xr_ab[0] + xl_ab[0] + e_ab[0]
                    m1 = xr_ab[1] + xl_ab[1] + e_ab[1]
                    a_lo = a_lo + jnp.maximum(m0, 0.2 * m0) * attb_v[wi, :]
                    a_hi = a_hi + jnp.maximum(m1, 0.2 * m1) * attb_v[64 + wi, :]
                    xr_lo.append(xr_ab[0])
                    xr_hi.append(xr_ab[1])
                w_lo = jnp.exp(a_lo)
                w_hi = jnp.exp(a_hi)
                for cc in range(C1):
                    wi = hp * C1 + cc
                    plsc.store_scatter(
                        o_r, [eix, jnp.full((LANES,), wi, jnp.int32)],
                        xr_lo[cc] * w_lo)
                    plsc.store_scatter(
                        o_r, [eix, jnp.full((LANES,), 64 + wi, jnp.int32)],
                        xr_hi[cc] * w_hi)
                plsc.store_scatter(
                    o_r, [eix, jnp.full((LANES,), 128 + hp, jnp.int32)], w_lo)
                plsc.store_scatter(
                    o_r, [eix, jnp.full((LANES,), 132 + hp, jnp.int32)], w_hi)
            return gcarry

        lax.fori_loop(0, _K // LANES, group_body, 0)

    _sc_pipeline(src1_hbm, dst1_hbm, (xr_hbm, xl_hbm), e1_hbm, zero_hbm,
                 out_hbm, ((si0, di0), (si1, di1)), dsc, (xr0, xl0, e0),
                 o0, acc, (isem0, isem1), gsem, ssem, compute, ())


def _conv2_sc_body(tsrc_hbm, tdst_hbm, e2s_hbm, src1_hbm, dst1_hbm, attb_hbm,
                   zero_hbm, out_hbm,
                   si0, di0, si1, di1, dsc, s0, d0, e0,
                   o0, attb_v, acc, isem0, isem1, gsem, ssem):
    pltpu.sync_copy(attb_hbm, attb_v)

    def compute():
        s_r, d_r, e_r = s0, d0, e0
        o_r = o0

        def unpk(ref, eix, wi):
            word = plsc.load_gather(ref, [eix, jnp.full((LANES,), wi, jnp.int32)])
            return plsc.unpack(plsc.bitcast(word, jnp.bfloat16),
                               format=plsc.PackFormat.INTERLEAVED)

        def group_body(g, gcarry):
            eix = lax.iota(jnp.int32, LANES) + g * LANES
            acc2 = jnp.zeros((LANES,), jnp.float32)
            accs = jnp.zeros((LANES,), jnp.float32)
            xr2_cs = []
            xrs_cs = []
            for cc in range(16):
                xr2_sk = unpk(s_r, eix, cc)
                xrs_pd = unpk(s_r, eix, 16 + cc)
                xl_ab = unpk(d_r, eix, cc)
                e_ab = unpk(e_r, eix, cc)
                m2 = xr2_sk[0] + xl_ab[0] + e_ab[0]
                ms = xrs_pd[0] + xl_ab[1] + e_ab[1] + xr2_sk[1]
                l2 = jnp.maximum(m2, 0.2 * m2)
                ls = jnp.maximum(ms, 0.2 * ms)
                acc2 = acc2 + l2 * attb_v[cc, :]
                accs = accs + ls * attb_v[16 + cc, :]
                xr2_cs.append(xr2_sk[0])
                xrs_cs.append(xrs_pd[0])
            w2 = jnp.exp(acc2)
            ws = jnp.exp(accs)
            for cc in range(16):
                cf0 = jnp.full((LANES,), cc, jnp.int32)
                cf1 = jnp.full((LANES,), 16 + cc, jnp.int32)
                plsc.store_scatter(o_r, [eix, cf0], xr2_cs[cc] * w2)
                plsc.store_scatter(o_r, [eix, cf1], xrs_cs[cc] * ws)
            plsc.store_scatter(o_r,
                               [eix, jnp.full((LANES,), 32, jnp.int32)], w2)
            plsc.store_scatter(o_r,
                               [eix, jnp.full((LANES,), 33, jnp.int32)], ws)
            return gcarry

        lax.fori_loop(0, _K // LANES, group_body, 0)

    _sc_pipeline(src1_hbm, dst1_hbm, (tsrc_hbm, tdst_hbm), e2s_hbm, zero_hbm,
                 out_hbm, ((si0, di0), (si1, di1)), dsc, (s0, d0, e0),
                 o0, acc, (isem0, isem1), gsem, ssem, compute, (o0,))


def _sc_mesh():
    return plsc.VectorSubcoreMesh(core_axis_name="c", subcore_axis_name="s",
                                  num_cores=NC, num_subcores=NS)


_SC_PARAMS = pltpu.CompilerParams(needs_layout_passes=False,
                                  use_tc_tiling_on_sc=False)

_IDX_SCRATCH = [pltpu.VMEM((_K,), jnp.int32)] * 5
_SEM_SCRATCH = [pltpu.SemaphoreType.DMA] * 4

_conv1_sc = pl.kernel(
    _conv1_sc_body,
    out_type=jax.ShapeDtypeStruct((NC * N, W1), jnp.float32),
    mesh=_sc_mesh(),
    compiler_params=_SC_PARAMS,
    scratch_types=_IDX_SCRATCH + [
        pltpu.VMEM((_K, 64), jnp.int32),
        pltpu.VMEM((_K, 64), jnp.int32),
        pltpu.VMEM((_K, 64), jnp.int32),
        pltpu.VMEM((_K, W1), jnp.float32),
        pltpu.VMEM((128, 16), jnp.float32),
        pltpu.VMEM_SHARED((N, W1), jnp.float32),
    ] + _SEM_SCRATCH,
)

_conv2_sc = pl.kernel(
    _conv2_sc_body,
    out_type=jax.ShapeDtypeStruct((NC * N, W2), jnp.float32),
    mesh=_sc_mesh(),
    compiler_params=_SC_PARAMS,
    scratch_types=_IDX_SCRATCH + [
        pltpu.VMEM((_K, 32), jnp.int32),
        pltpu.VMEM((_K, 16), jnp.int32),
        pltpu.VMEM((_K, 16), jnp.int32),
        pltpu.VMEM((_K, W2), jnp.float32),
        pltpu.VMEM((32, 16), jnp.float32),
        pltpu.VMEM_SHARED((N, W2), jnp.float32),
    ] + _SEM_SCRATCH,
)


@jax.jit
def kernel(x, edge_index, edge_attr, Wl1, Wr1, We1, att1, b1, Wl2, Wr2, We2,
           att2, b2, Wls, Wrs, Wes, atts, Wskip, bs, Wlin, blin, gamma, beta):
    f32 = jnp.float32
    src1 = edge_index[0]
    dst1 = edge_index[1]

    xl1, xr1 = pl.pallas_call(
        _proj1_body,
        grid=(10,),
        in_specs=[
            pl.BlockSpec((1000, 128), lambda i: (i, 0)),
            pl.BlockSpec((128, 128), lambda i: (0, 0)),
            pl.BlockSpec((128, 128), lambda i: (0, 0)),
        ],
        out_specs=[
            pl.BlockSpec((1000, 64), lambda i: (i, 0)),
            pl.BlockSpec((1000, 64), lambda i: (i, 0)),
        ],
        out_shape=[
            jax.ShapeDtypeStruct((N, 64), jnp.int32),
            jax.ShapeDtypeStruct((N, 64), jnp.int32),
        ],
    )(x, Wl1, Wr1)
    xl1i, xr1i = xl1, xr1

    We2s = jnp.concatenate([We2, Wes], axis=1)
    e1, e2s = pl.pallas_call(
        _edge_proj_body,
        grid=(320,),
        in_specs=[
            pl.BlockSpec((1000, 16), lambda i: (i, 0)),
            pl.BlockSpec((16, 128), lambda i: (0, 0)),
            pl.BlockSpec((16, 32), lambda i: (0, 0)),
        ],
        out_specs=[
            pl.BlockSpec((1000, 64), lambda i: (i, 0)),
            pl.BlockSpec((1000, 16), lambda i: (i, 0)),
        ],
        out_shape=[
            jax.ShapeDtypeStruct((E, 64), jnp.int32),
            jax.ShapeDtypeStruct((E, 16), jnp.int32),
        ],
    )(edge_attr, We1, We2s)
    e1i, e2si = e1, e2s

    attb1 = jnp.broadcast_to(att1.reshape(H1 * C1, 1), (H1 * C1, 16))
    zeros1 = jnp.zeros((N, W1), f32)
    p1 = _conv1_sc(xr1i, xl1i, e1i, src1, dst1, attb1, zeros1)

    sel = jnp.repeat(jnp.eye(H1, dtype=f32), C1, axis=1)
    Wsrc = jnp.concatenate([Wr2, Wrs], axis=1)
    Wdst = jnp.concatenate([Wl2, Wls], axis=1)
    tsrc, tdst = pl.pallas_call(
        _stage_c_body,
        grid=(10,),
        in_specs=[
            pl.BlockSpec((1000, W1), lambda i: (i, 0)),
            pl.BlockSpec((1000, W1), lambda i: (i + 10, 0)),
            pl.BlockSpec((1000, 64), lambda i: (i, 0)),
            pl.BlockSpec((1, 128), lambda i: (0, 0)),
            pl.BlockSpec((8, 128), lambda i: (0, 0)),
            pl.BlockSpec((128, 32), lambda i: (0, 0)),
            pl.BlockSpec((128, 32), lambda i: (0, 0)),
            pl.BlockSpec((128, 16), lambda i: (0, 0)),
        ],
        out_specs=[
            pl.BlockSpec((1000, 32), lambda i: (i, 0)),
            pl.BlockSpec((1000, 16), lambda i: (i, 0)),
        ],
        out_shape=[
            jax.ShapeDtypeStruct((N, 32), jnp.int32),
            jax.ShapeDtypeStruct((N, 16), jnp.int32),
        ],
    )(p1, p1, xl1, b1.reshape(1, 128), sel, Wsrc, Wdst, Wskip)
    tsrci, tdsti = tsrc, tdst

    attb2 = jnp.broadcast_to(
        jnp.concatenate([att2.reshape(16), atts.reshape(16)]).reshape(32, 1),
        (32, 16))
    zeros2 = jnp.zeros((N, W2), f32)
    p2 = _conv2_sc(tsrci, tdsti, e2si, src1, dst1, attb2, zeros2)

    y = pl.pallas_call(
        _stage_e_body,
        grid=(10,),
        in_specs=[
            pl.BlockSpec((1000, W2), lambda i: (i, 0)),
            pl.BlockSpec((1000, W2), lambda i: (i + 10, 0)),
            pl.BlockSpec((1, 16), lambda i: (0, 0)),
            pl.BlockSpec((1, 16), lambda i: (0, 0)),
            pl.BlockSpec((16, 16), lambda i: (0, 0)),
            pl.BlockSpec((1, 16), lambda i: (0, 0)),
            pl.BlockSpec((1, 16), lambda i: (0, 0)),
            pl.BlockSpec((1, 16), lambda i: (0, 0)),
        ],
        out_specs=pl.BlockSpec((1000, 16), lambda i: (i, 0)),
        out_shape=jax.ShapeDtypeStruct((N, 16), f32),
    )(p2, p2, b2.reshape(1, 16), bs.reshape(1, 16), Wlin,
      blin.reshape(1, 16), gamma.reshape(1, 16), beta.reshape(1, 16))

    return jnp.stack((y,))
